# baseline (speedup 1.0000x reference)
import jax
import jax.numpy as jnp
from jax.experimental import pallas as pl
from jax.experimental.pallas import tpu as pltpu

B = 8
N = 2048
IC = 512
NT = N // IC
KC = N // NT


def _count_chunk(srow, scol_ref, c):
    sk = scol_ref[0, pl.ds(c * KC, KC), :]
    gt = sk > srow
    eq = sk == srow
    kio = jax.lax.broadcasted_iota(jnp.int32, (KC, N), 0) + c * KC
    jio = jax.lax.broadcasted_iota(jnp.int32, (KC, N), 1)
    cnt = jnp.where(gt, 1, 0) + jnp.where(eq, 65536, 0)
    bc = jnp.where(eq & (kio < jio), 1, 0)
    tot = jnp.sum(cnt, axis=0, keepdims=True)
    bef = jnp.sum(bc, axis=0, keepdims=True)
    return tot, bef


def _fused_body(srow0_ref, scol0_ref, srow1_ref, scol1_ref, out_ref,
                tot_ref, bef_ref):
    b = pl.program_id(0)
    t = pl.program_id(1)

    @pl.when((b == 0) & (t == 0))
    def _():
        srow = srow0_ref[0]
        tot = None
        bef = None
        for c in range(NT):
            tc, bc = _count_chunk(srow, scol0_ref, c)
            tot = tc if tot is None else tot + tc
            bef = bc if bef is None else bef + bc
        tot_ref[0] = tot
        bef_ref[0] = bef

    nslot = (b + 1) % 2
    tc, bc = _count_chunk(srow1_ref[0], scol1_ref, t)

    @pl.when(t == 0)
    def _():
        tot_ref[nslot] = tc
        bef_ref[nslot] = bc

    @pl.when(t > 0)
    def _():
        tot_ref[nslot] = tot_ref[nslot] + tc
        bef_ref[nslot] = bef_ref[nslot] + bc

    slot = b % 2
    tot = tot_ref[slot]
    lo = tot & 65535
    hi = lo + (tot >> 16)
    valid = bef_ref[slot] == 0
    iio = jax.lax.broadcasted_iota(jnp.int32, (IC, N), 0) + t * IC
    p = (iio >= lo) & (iio < hi) & valid
    out_ref[0] = p.astype(jnp.float32)


def kernel(s):
    sr = s.reshape(B, 1, N)
    sc = s.reshape(B, N, 1)
    return pl.pallas_call(
        _fused_body,
        grid=(B, NT),
        in_specs=[
            pl.BlockSpec((1, 1, N), lambda b, t: (b, 0, 0)),
            pl.BlockSpec((1, N, 1), lambda b, t: (b, 0, 0)),
            pl.BlockSpec((1, 1, N), lambda b, t: (jnp.minimum(b + 1, B - 1), 0, 0)),
            pl.BlockSpec((1, N, 1), lambda b, t: (jnp.minimum(b + 1, B - 1), 0, 0)),
        ],
        out_specs=pl.BlockSpec((1, IC, N), lambda b, t: (b, t, 0)),
        out_shape=jax.ShapeDtypeStruct((B, N, N), jnp.float32),
        scratch_shapes=[
            pltpu.VMEM((2, 1, N), jnp.int32),
            pltpu.VMEM((2, 1, N), jnp.int32),
        ],
    )(sr, sc, sr, sc)


# speedup vs baseline: 1.1720x; 1.1720x over previous
import jax
import jax.numpy as jnp
from jax.experimental import pallas as pl
from jax.experimental.pallas import tpu as pltpu

B = 8
N = 2048
KC = 256
IC = 512


def _fused_body(srow_ref, scol_ref, out_ref, lo_ref, m_ref):
    t = pl.program_id(1)

    @pl.when(t == 0)
    def _():
        srow = srow_ref[0]
        acc = None
        bacc = None
        for c in range(N // KC):
            sk = scol_ref[0, c * KC:(c + 1) * KC, :]
            gt = sk > srow
            eq = sk == srow
            kio = jax.lax.broadcasted_iota(jnp.int32, (KC, N), 0) + c * KC
            jio = jax.lax.broadcasted_iota(jnp.int32, (KC, N), 1)
            cnt = jnp.where(gt, 1, jnp.where(eq, 65536, 0))
            bc = jnp.where(eq & (kio < jio), 1, 0)
            acc = cnt if acc is None else acc + cnt
            bacc = bc if bacc is None else bacc + bc
        tot = jnp.sum(acc, axis=0, keepdims=True)
        before = jnp.sum(bacc, axis=0, keepdims=True)
        lo_ref[...] = tot & 65535
        m_ref[...] = jnp.where(before == 0, tot >> 16, 0)

    lo = lo_ref[...]
    mv = m_ref[...].astype(jnp.uint32)
    iio = jax.lax.broadcasted_iota(jnp.int32, (IC, N), 0) + t * IC
    p = (iio - lo).astype(jnp.uint32) < mv
    out_ref[0] = jnp.where(p, 1.0, 0.0)


def kernel(s):
    return pl.pallas_call(
        _fused_body,
        grid=(B, N // IC),
        in_specs=[
            pl.BlockSpec((1, 1, N), lambda b, t: (b, 0, 0)),
            pl.BlockSpec((1, N, 1), lambda b, t: (b, 0, 0)),
        ],
        out_specs=pl.BlockSpec((1, IC, N), lambda b, t: (b, t, 0)),
        out_shape=jax.ShapeDtypeStruct((B, N, N), jnp.float32),
        scratch_shapes=[
            pltpu.VMEM((1, N), jnp.int32),
            pltpu.VMEM((1, N), jnp.int32),
        ],
    )(s.reshape(B, 1, N), s.reshape(B, N, 1))


# speedup vs baseline: 1.2607x; 1.0757x over previous
import jax
import jax.numpy as jnp
from jax.experimental import pallas as pl
from jax.experimental.pallas import tpu as pltpu

B = 8
N = 2048
KC = 256
IC = 512


def _fused_body(srow_ref, scol_ref, out_ref, lo_ref, m_ref):
    t = pl.program_id(1)

    @pl.when(t == 0)
    def _():
        srow = srow_ref[0]
        ones_row = jnp.ones((1, KC), jnp.bfloat16)
        tot = None
        bef = None
        for c in range(N // KC):
            sk = scol_ref[0, c * KC:(c + 1) * KC, :]
            gt = sk > srow
            eq = sk == srow
            kio = jax.lax.broadcasted_iota(jnp.int32, (KC, N), 0) + c * KC
            jio = jax.lax.broadcasted_iota(jnp.int32, (KC, N), 1)
            cnt = jnp.where(gt, 1.0, jnp.where(eq, 4096.0, 0.0)).astype(
                jnp.bfloat16)
            bc = jnp.where(eq & (kio < jio), 1.0, 0.0).astype(jnp.bfloat16)
            tc = jax.lax.dot_general(
                ones_row, cnt, (((1,), (0,)), ((), ())),
                preferred_element_type=jnp.float32,
            )
            bcs = jax.lax.dot_general(
                ones_row, bc, (((1,), (0,)), ((), ())),
                preferred_element_type=jnp.float32,
            )
            tot = tc if tot is None else tot + tc
            bef = bcs if bef is None else bef + bcs
        toti = tot.astype(jnp.int32)
        lo_ref[...] = toti & 4095
        m_ref[...] = jnp.where(bef == 0.0, toti >> 12, 0)

    lo = lo_ref[...]
    mv = m_ref[...].astype(jnp.uint32)
    iio = jax.lax.broadcasted_iota(jnp.int32, (IC, N), 0) + t * IC
    p = (iio - lo).astype(jnp.uint32) < mv
    out_ref[0] = jnp.where(p, 1.0, 0.0)


def kernel(s):
    return pl.pallas_call(
        _fused_body,
        grid=(B, N // IC),
        in_specs=[
            pl.BlockSpec((1, 1, N), lambda b, t: (b, 0, 0)),
            pl.BlockSpec((1, N, 1), lambda b, t: (b, 0, 0)),
        ],
        out_specs=pl.BlockSpec((1, IC, N), lambda b, t: (b, t, 0)),
        out_shape=jax.ShapeDtypeStruct((B, N, N), jnp.float32),
        scratch_shapes=[
            pltpu.VMEM((1, N), jnp.int32),
            pltpu.VMEM((1, N), jnp.int32),
        ],
    )(s.reshape(B, 1, N), s.reshape(B, N, 1))


# speedup vs baseline: 1.4899x; 1.1818x over previous
import jax
import jax.numpy as jnp
from jax.experimental import pallas as pl
from jax.experimental.pallas import tpu as pltpu

B = 8
N = 2048
KC = 512
IC = 1024


def _fused_body(srow_ref, scol_ref, out_ref, lo_ref, m_ref):
    t = pl.program_id(1)

    @pl.when(t == 0)
    def _():
        srow = srow_ref[0]
        ones_row = jnp.ones((1, KC), jnp.bfloat16)
        tot = None
        bef = None
        for c in range(N // KC):
            sk = scol_ref[0, c * KC:(c + 1) * KC, :]
            gt = sk > srow
            eq = sk == srow
            kio = jax.lax.broadcasted_iota(jnp.int32, (KC, N), 0) + c * KC
            jio = jax.lax.broadcasted_iota(jnp.int32, (KC, N), 1)
            cnt = jnp.where(gt, 1.0, jnp.where(eq, 4096.0, 0.0)).astype(
                jnp.bfloat16)
            bc = jnp.where(eq & (kio < jio), 1.0, 0.0).astype(jnp.bfloat16)
            tc = jax.lax.dot_general(
                ones_row, cnt, (((1,), (0,)), ((), ())),
                preferred_element_type=jnp.float32,
            )
            bcs = jax.lax.dot_general(
                ones_row, bc, (((1,), (0,)), ((), ())),
                preferred_element_type=jnp.float32,
            )
            tot = tc if tot is None else tot + tc
            bef = bcs if bef is None else bef + bcs
        toti = tot.astype(jnp.int32)
        lo_ref[...] = toti & 4095
        m_ref[...] = jnp.where(bef == 0.0, toti >> 12, 0)

    lo = lo_ref[...]
    mv = m_ref[...].astype(jnp.uint32)
    iio = jax.lax.broadcasted_iota(jnp.int32, (IC, N), 0) + t * IC
    p = (iio - lo).astype(jnp.uint32) < mv
    out_ref[0] = jnp.where(p, 1.0, 0.0)


def kernel(s):
    return pl.pallas_call(
        _fused_body,
        grid=(B, N // IC),
        in_specs=[
            pl.BlockSpec((1, 1, N), lambda b, t: (b, 0, 0)),
            pl.BlockSpec((1, N, 1), lambda b, t: (b, 0, 0)),
        ],
        out_specs=pl.BlockSpec((1, IC, N), lambda b, t: (b, t, 0)),
        out_shape=jax.ShapeDtypeStruct((B, N, N), jnp.float32),
        scratch_shapes=[
            pltpu.VMEM((1, N), jnp.int32),
            pltpu.VMEM((1, N), jnp.int32),
        ],
    )(s.reshape(B, 1, N), s.reshape(B, N, 1))


# speedup vs baseline: 1.6611x; 1.1149x over previous
import jax
import jax.numpy as jnp
from jax.experimental import pallas as pl
from jax.experimental.pallas import tpu as pltpu

B = 8
N = 2048
KC = 1024
IC = 2048


def _fused_body(srow_ref, scol_ref, out_ref, lo_ref, m_ref):
    t = pl.program_id(1)

    @pl.when(t == 0)
    def _():
        srow = srow_ref[0]
        ones_row = jnp.ones((1, KC), jnp.bfloat16)
        tot = None
        bef = None
        for c in range(N // KC):
            sk = scol_ref[0, c * KC:(c + 1) * KC, :]
            gt = sk > srow
            eq = sk == srow
            kio = jax.lax.broadcasted_iota(jnp.int32, (KC, N), 0) + c * KC
            jio = jax.lax.broadcasted_iota(jnp.int32, (KC, N), 1)
            cnt = jnp.where(gt, 1.0, jnp.where(eq, 4096.0, 0.0)).astype(
                jnp.bfloat16)
            bc = jnp.where(eq & (kio < jio), 1.0, 0.0).astype(jnp.bfloat16)
            tc = jax.lax.dot_general(
                ones_row, cnt, (((1,), (0,)), ((), ())),
                preferred_element_type=jnp.float32,
            )
            bcs = jax.lax.dot_general(
                ones_row, bc, (((1,), (0,)), ((), ())),
                preferred_element_type=jnp.float32,
            )
            tot = tc if tot is None else tot + tc
            bef = bcs if bef is None else bef + bcs
        toti = tot.astype(jnp.int32)
        lo_ref[...] = toti & 4095
        m_ref[...] = jnp.where(bef == 0.0, toti >> 12, 0)

    lo = lo_ref[...]
    mv = m_ref[...].astype(jnp.uint32)
    iio = jax.lax.broadcasted_iota(jnp.int32, (IC, N), 0) + t * IC
    p = (iio - lo).astype(jnp.uint32) < mv
    out_ref[0] = jnp.where(p, 1.0, 0.0)


def kernel(s):
    return pl.pallas_call(
        _fused_body,
        grid=(B, N // IC),
        in_specs=[
            pl.BlockSpec((1, 1, N), lambda b, t: (b, 0, 0)),
            pl.BlockSpec((1, N, 1), lambda b, t: (b, 0, 0)),
        ],
        out_specs=pl.BlockSpec((1, IC, N), lambda b, t: (b, t, 0)),
        out_shape=jax.ShapeDtypeStruct((B, N, N), jnp.float32),
        scratch_shapes=[
            pltpu.VMEM((1, N), jnp.int32),
            pltpu.VMEM((1, N), jnp.int32),
        ],
    )(s.reshape(B, 1, N), s.reshape(B, N, 1))
